# baseline (speedup 1.0000x reference)
import functools

import jax
import jax.numpy as jnp
from jax import lax
from jax.experimental import pallas as pl
from jax.experimental.pallas import tpu as pltpu
from jax.experimental.pallas import tpu_sc as plsc

VOCAB = 1_000_000
EMB = 32
HALF = EMB // 2
B = 16384
T = 50
CHUNK = 1024
N_CHUNKS = B // CHUNK
N_UNITS = T * N_CHUNKS
RW = 128
NSTREAM = CHUNK // RW
VFULL = VOCAB // CHUNK
VTAIL = VOCAB - VFULL * CHUNK


@functools.lru_cache(maxsize=1)
def _build():
    info = plsc.get_sparse_core_info()
    ns = info.num_subcores

    mesh = plsc.VectorSubcoreMesh(core_axis_name="c", subcore_axis_name="s")

    @functools.partial(
        pl.kernel,
        mesh=mesh,
        compiler_params=pltpu.CompilerParams(
            use_tc_tiling_on_sc=False, needs_layout_passes=False
        ),
        out_type=(
            jax.ShapeDtypeStruct((T, EMB, B), jnp.float32),
            jax.ShapeDtypeStruct((2, VOCAB, HALF), jnp.float32),
        ),
        scratch_types=[
            pltpu.VMEM((HALF, CHUNK), jnp.float32),
            pltpu.VMEM((CHUNK, HALF), jnp.float32),
            pltpu.VMEM((CHUNK,), jnp.int32),
            pltpu.VMEM((CHUNK, HALF), jnp.float32),
            pltpu.VMEM((HALF, CHUNK), jnp.float32),
            pltpu.SemaphoreType.DMA,
        ],
    )
    def emb_kernel(
        tt_hbm, xt_hbm, out_hbm, scr_hbm,
        tsrc_v, tdst_v, idx_v, rows_v, outb_v, sem,
    ):
        c = lax.axis_index("c")
        sid = lax.axis_index("s")
        e0 = c * HALF
        lanes = lax.iota(jnp.int32, 16)

        def diag_transpose(src, dst, ngroups):
            @plsc.parallel_loop(0, ngroups, unroll=4)
            def _(jj):
                col16 = jj * 16 + lanes
                erow = lanes
                for _s in range(HALF):
                    vals = plsc.load_gather(src, [erow, col16])
                    plsc.store_scatter(dst, [col16, erow], vals)
                    erow = jnp.bitwise_and(erow + 1, HALF - 1)

        def a_body(i, _):
            v0 = (sid * (VFULL // ns) + i) * CHUNK
            pltpu.sync_copy(tt_hbm.at[pl.ds(e0, HALF), pl.ds(v0, CHUNK)],
                            tsrc_v)
            diag_transpose(tsrc_v, tdst_v, CHUNK // 16)
            pltpu.sync_copy(tdst_v, scr_hbm.at[c, pl.ds(v0, CHUNK)])
            return 0

        lax.fori_loop(0, VFULL // ns, a_body, 0)

        @pl.when(sid == ns - 1)
        def _():
            v0 = VFULL * CHUNK
            pltpu.sync_copy(tt_hbm.at[pl.ds(e0, HALF), pl.ds(v0, VTAIL)],
                            tsrc_v.at[:, pl.ds(0, VTAIL)])
            diag_transpose(tsrc_v, tdst_v, VTAIL // 16)
            pltpu.sync_copy(tdst_v.at[pl.ds(0, VTAIL)],
                            scr_hbm.at[c, pl.ds(v0, VTAIL)])

        plsc.subcore_barrier()

        def b_body(i, _):
            u = sid + i * ns
            t = u // N_CHUNKS
            b0 = (u % N_CHUNKS) * CHUNK
            pltpu.sync_copy(xt_hbm.at[t, pl.ds(b0, CHUNK)], idx_v)
            copies = [
                pltpu.async_copy(
                    scr_hbm.at[c].at[idx_v.at[pl.ds(j * RW, RW)]],
                    rows_v.at[pl.ds(j * RW, RW)],
                    sem,
                )
                for j in range(NSTREAM)
            ]
            for cp in copies:
                cp.wait()

            @plsc.parallel_loop(0, CHUNK // 16, unroll=4)
            def _(jj):
                row16 = jj * 16 + lanes
                ecol = lanes
                for _s in range(HALF):
                    vals = plsc.load_gather(rows_v, [row16, ecol])
                    plsc.store_scatter(outb_v, [ecol, row16], vals)
                    ecol = jnp.bitwise_and(ecol + 1, HALF - 1)

            pltpu.sync_copy(
                outb_v, out_hbm.at[t, pl.ds(e0, HALF), pl.ds(b0, CHUNK)]
            )
            return 0

        lax.fori_loop(0, N_UNITS // ns, b_body, 0)

    return emb_kernel


def kernel(x, table):
    tt = table.T
    xt = x.T
    outp, _ = _build()(tt, xt)
    return jnp.transpose(outp, (2, 0, 1))


# speedup vs baseline: 4.2303x; 4.2303x over previous
import functools

import jax
import jax.numpy as jnp
from jax import lax
from jax.experimental import pallas as pl
from jax.experimental.pallas import tpu as pltpu
from jax.experimental.pallas import tpu_sc as plsc

VOCAB = 1_000_000
EMB = 32
B = 16384
T = 50
CHUNK = 1024
N_CHUNKS = B // CHUNK
N_UNITS = T * N_CHUNKS
RW = 128
NSTREAM = CHUNK // RW


@functools.lru_cache(maxsize=1)
def _build():
    info = plsc.get_sparse_core_info()
    nc, ns = info.num_cores, info.num_subcores
    nw = nc * ns
    units_per_w = N_UNITS // nw

    mesh = plsc.VectorSubcoreMesh(core_axis_name="c", subcore_axis_name="s")

    @functools.partial(
        pl.kernel,
        mesh=mesh,
        compiler_params=pltpu.CompilerParams(
            use_tc_tiling_on_sc=False, needs_layout_passes=False
        ),
        out_type=jax.ShapeDtypeStruct((T, EMB, B), jnp.float32),
        scratch_types=[
            pltpu.VMEM((CHUNK,), jnp.int32),
            pltpu.VMEM((CHUNK, EMB), jnp.float32),
            pltpu.VMEM((CHUNK, EMB), jnp.float32),
            pltpu.VMEM((EMB, CHUNK), jnp.float32),
            pltpu.SemaphoreType.DMA,
            pltpu.SemaphoreType.DMA,
        ],
    )
    def emb_kernel(
        table_hbm, xt_hbm, out_hbm, idx_v, rows_a, rows_b, outb_v, sem_a, sem_b
    ):
        wid = lax.axis_index("s") * nc + lax.axis_index("c")
        lanes = lax.iota(jnp.int32, 16)

        def fire(k, rows, sem):
            u = wid + k * nw
            t = u // N_CHUNKS
            b0 = (u % N_CHUNKS) * CHUNK
            pltpu.sync_copy(xt_hbm.at[t, pl.ds(b0, CHUNK)], idx_v)
            return [
                pltpu.async_copy(
                    table_hbm.at[idx_v.at[pl.ds(j * RW, RW)]],
                    rows.at[pl.ds(j * RW, RW)],
                    sem,
                )
                for j in range(NSTREAM)
            ]

        def drain(rows, sem):
            for j in range(NSTREAM):
                pltpu.make_async_copy(
                    table_hbm.at[idx_v.at[pl.ds(j * RW, RW)]],
                    rows.at[pl.ds(j * RW, RW)],
                    sem,
                ).wait()

        def finish(k, rows):
            u = wid + k * nw
            t = u // N_CHUNKS
            b0 = (u % N_CHUNKS) * CHUNK

            @plsc.parallel_loop(0, CHUNK // 16, unroll=4)
            def tr_body(jj):
                row16 = jj * 16 + lanes
                ecol = lanes
                for _ in range(EMB):
                    vals = plsc.load_gather(rows, [row16, ecol])
                    plsc.store_scatter(outb_v, [ecol, row16], vals)
                    ecol = jnp.bitwise_and(ecol + 1, EMB - 1)

            pltpu.sync_copy(outb_v, out_hbm.at[t, :, pl.ds(b0, CHUNK)])

        fire(0, rows_a, sem_a)

        def pair_body(p, _):
            drain(rows_a, sem_a)
            fire(2 * p + 1, rows_b, sem_b)
            finish(2 * p, rows_a)
            drain(rows_b, sem_b)
            fire(2 * p + 2, rows_a, sem_a)
            finish(2 * p + 1, rows_b)
            return 0

        lax.fori_loop(0, (units_per_w - 1) // 2, pair_body, 0)
        drain(rows_a, sem_a)
        finish(units_per_w - 1, rows_a)

    return emb_kernel


def kernel(x, table):
    xt = x.T
    outp = _build()(table, xt)
    return jnp.transpose(outp, (2, 0, 1))


# speedup vs baseline: 4.2591x; 1.0068x over previous
import functools

import jax
import jax.numpy as jnp
from jax import lax
from jax.experimental import pallas as pl
from jax.experimental.pallas import tpu as pltpu
from jax.experimental.pallas import tpu_sc as plsc

VOCAB = 1_000_000
EMB = 32
B = 16384
T = 50
RW = 128


@functools.lru_cache(maxsize=1)
def _build():
    info = plsc.get_sparse_core_info()
    nc, ns = info.num_cores, info.num_subcores
    nw = nc * ns
    blk = B // nw
    nstream = blk // RW

    mesh = plsc.VectorSubcoreMesh(core_axis_name="c", subcore_axis_name="s")

    @functools.partial(
        pl.kernel,
        mesh=mesh,
        compiler_params=pltpu.CompilerParams(
            use_tc_tiling_on_sc=False, needs_layout_passes=False
        ),
        out_type=jax.ShapeDtypeStruct((T, EMB, B), jnp.float32),
        scratch_types=[
            pltpu.VMEM((blk, T), jnp.int32),
            pltpu.VMEM((blk,), jnp.int32),
            pltpu.VMEM((blk, EMB), jnp.float32),
            pltpu.VMEM((blk, EMB), jnp.float32),
            pltpu.VMEM((EMB, blk), jnp.float32),
            pltpu.SemaphoreType.DMA,
            pltpu.SemaphoreType.DMA,
        ],
    )
    def emb_kernel(
        table_hbm, x_hbm, out_hbm,
        xb_v, idx_v, rows_a, rows_b, outb_v, sem_a, sem_b,
    ):
        wid = lax.axis_index("s") * nc + lax.axis_index("c")
        b0 = wid * blk
        lanes = lax.iota(jnp.int32, 16)

        pltpu.sync_copy(x_hbm.at[pl.ds(b0, blk)], xb_v)

        def fire(t, rows, sem):
            @plsc.parallel_loop(0, blk // 16, unroll=4)
            def _(jj):
                r16 = jj * 16 + lanes
                idx_v[pl.ds(jj * 16, 16)] = plsc.load_gather(
                    xb_v, [r16, jnp.full((16,), t, jnp.int32)]
                )

            return [
                pltpu.async_copy(
                    table_hbm.at[idx_v.at[pl.ds(j * RW, RW)]],
                    rows.at[pl.ds(j * RW, RW)],
                    sem,
                )
                for j in range(nstream)
            ]

        def drain(rows, sem):
            for j in range(nstream):
                pltpu.make_async_copy(
                    table_hbm.at[idx_v.at[pl.ds(j * RW, RW)]],
                    rows.at[pl.ds(j * RW, RW)],
                    sem,
                ).wait()

        def finish(t, rows):
            @plsc.parallel_loop(0, blk // 16, unroll=4)
            def _(jj):
                row16 = jj * 16 + lanes
                ecol = lanes
                for _s in range(EMB):
                    vals = plsc.load_gather(rows, [row16, ecol])
                    plsc.store_scatter(outb_v, [ecol, row16], vals)
                    ecol = jnp.bitwise_and(ecol + 1, EMB - 1)

            pltpu.sync_copy(outb_v, out_hbm.at[t, :, pl.ds(b0, blk)])

        fire(0, rows_a, sem_a)

        def pair_body(p, _):
            drain(rows_a, sem_a)
            fire(2 * p + 1, rows_b, sem_b)
            finish(2 * p, rows_a)
            drain(rows_b, sem_b)
            fire(2 * p + 2, rows_a, sem_a)
            finish(2 * p + 1, rows_b)
            return 0

        lax.fori_loop(0, T // 2 - 1, pair_body, 0)
        drain(rows_a, sem_a)
        fire(T - 1, rows_b, sem_b)
        finish(T - 2, rows_a)
        drain(rows_b, sem_b)
        finish(T - 1, rows_b)

    return emb_kernel


def kernel(x, table):
    outp = _build()(table, x)
    return jnp.transpose(outp, (2, 0, 1))
